# baseline (speedup 1.0000x reference)
import functools

import jax
import jax.numpy as jnp
from jax import lax
from jax.experimental import pallas as pl
from jax.experimental.pallas import tpu as pltpu
from jax.experimental.pallas import tpu_sc as plsc

VOCAB = 1000000
DIM = 64
NB = 4096
NL = 200

NC = 2
NS = 16
NW = NC * NS
N_IDX = NB * NL
PER_W = N_IDX // NW
CHUNK = 128
N_CHUNK = PER_W // CHUNK
NBUF = 4
NROUNDS = N_CHUNK // NBUF


def _make_gather():
  mesh = plsc.VectorSubcoreMesh(core_axis_name="c", subcore_axis_name="s")

  @functools.partial(
      pl.kernel,
      mesh=mesh,
      out_type=jax.ShapeDtypeStruct((N_IDX, DIM), jnp.float32),
      scratch_types=[
          pltpu.VMEM((N_CHUNK, CHUNK), jnp.int32),
          pltpu.VMEM((NBUF, CHUNK, DIM), jnp.float32),
      ] + [pltpu.SemaphoreType.DMA] * NBUF,
      compiler_params=pltpu.CompilerParams(use_tc_tiling_on_sc=False),
  )
  def k(idx_hbm, table_hbm, out_hbm, idx_v, rows_v, s0, s1, s2, s3):
    sems = [s0, s1, s2, s3]
    wid = lax.axis_index("s") * NC + lax.axis_index("c")
    base = wid * PER_W
    pltpu.sync_copy(idx_hbm.at[wid], idx_v)

    for b in range(NBUF):
      pltpu.async_copy(table_hbm.at[idx_v.at[b]], rows_v.at[b], sems[b])

    def round_body(i, carry):
      for b in range(NBUF):
        j = i * NBUF + b
        pltpu.make_async_copy(
            table_hbm.at[idx_v.at[b]], rows_v.at[b], sems[b]).wait()
        pltpu.sync_copy(rows_v.at[b],
                        out_hbm.at[pl.ds(base + j * CHUNK, CHUNK)])
        pltpu.async_copy(
            table_hbm.at[idx_v.at[j + NBUF]], rows_v.at[b], sems[b])
      return carry

    lax.fori_loop(0, NROUNDS - 1, round_body, 0)

    for b in range(NBUF):
      j = (NROUNDS - 1) * NBUF + b
      pltpu.make_async_copy(
          table_hbm.at[idx_v.at[b]], rows_v.at[b], sems[b]).wait()
      pltpu.sync_copy(rows_v.at[b],
                      out_hbm.at[pl.ds(base + j * CHUNK, CHUNK)])

  return k


_gather = _make_gather()


def kernel(input_ids, table):
  idx = input_ids.astype(jnp.int32).reshape(NW, N_CHUNK, CHUNK)
  out = _gather(idx, table)
  return out.reshape(NB, NL, DIM)
